# baseline (speedup 1.0000x reference)
import jax
import jax.numpy as jnp
from jax import lax
from jax.experimental import pallas as pl
from jax.experimental.pallas import tpu as pltpu
from jax.experimental.pallas import tpu_sc as plsc

B = 16384
F = 26
V = 100000
D = 16
FD = F * D
H0 = 1024
H1 = 512

NC, NS, L = 2, 16, 16
NW = NC * NS
PPW = FD // NW
GCH = 4096


def _plane_body(embt, wlin, xt, featt, wvt, plane_v, xv, ob0, ob1,
                sem0, sem1):
    wid = lax.axis_index("s") * NC + lax.axis_index("c")
    obs, sems = (ob0, ob1), (sem0, sem1)
    pending = [None, None]

    def extract(fd, f, load_x, out_hbm, table_hbm):
        @pl.when(load_x)
        def _():
            pltpu.sync_copy(xt.at[pl.ds(f * B, B)], xv)

        pltpu.sync_copy(table_hbm.at[fd], plane_v)

        for c in range(B // GCH):
            slot = c % 2
            ob = obs[slot]
            if pending[slot] is not None:
                pending[slot].wait()

            @plsc.parallel_loop(0, GCH // L, unroll=8)
            def grp(g):
                idx = xv[pl.ds(c * GCH + g * L, L)]
                ob[pl.ds(g * L, L)] = plsc.load_gather(plane_v, [idx])
            cp = pltpu.make_async_copy(
                ob, out_hbm.at[fd, pl.ds(c * GCH, GCH)], sems[slot])
            cp.start()
            pending[slot] = cp

    def drain():
        for slot in range(2):
            if pending[slot] is not None:
                pending[slot].wait()
                pending[slot] = None

    for k in range(PPW):
        fd = wid * PPW + k
        load_x = (fd % D == 0) if k else (fd == fd)
        extract(fd, fd // D, load_x, featt, embt)
    drain()

    @pl.when(wid < F)
    def _():
        extract(wid, wid, wid == wid, wvt, wlin)
        drain()


@jax.jit
def _sc_planes(embt, wlin, xt):
    mesh = plsc.VectorSubcoreMesh(core_axis_name="c", subcore_axis_name="s")
    return pl.kernel(
        _plane_body,
        out_type=(
            jax.ShapeDtypeStruct((FD, B), jnp.float32),
            jax.ShapeDtypeStruct((F, B), jnp.float32),
        ),
        mesh=mesh,
        compiler_params=pltpu.CompilerParams(needs_layout_passes=False),
        scratch_types=[
            pltpu.VMEM((V,), jnp.float32),
            pltpu.VMEM((B,), jnp.int32),
            pltpu.VMEM((GCH,), jnp.float32),
            pltpu.VMEM((GCH,), jnp.float32),
            pltpu.SemaphoreType.DMA,
            pltpu.SemaphoreType.DMA,
        ],
        name="deepfm_plane_gather",
    )(embt, wlin, xt)


BBT = 2048


def _tc_body(ft_ref, wv_ref, w0t_ref, b0_ref, w1t_ref, b1_ref, w2t_ref,
             b2b_ref, out_ref):
    ft = ft_ref[...]
    sel = (lax.broadcasted_iota(jnp.int32, (D, FD), 1) % D
           == lax.broadcasted_iota(jnp.int32, (D, FD), 0)
           ).astype(jnp.float32)
    s = jnp.dot(sel, ft, preferred_element_type=jnp.float32)
    sq = jnp.dot(sel, ft * ft, preferred_element_type=jnp.float32)
    inter = 0.5 * jnp.sum(s * s - sq, axis=0, keepdims=True)

    lin = jnp.sum(wv_ref[...], axis=0, keepdims=True)

    h = jnp.maximum(
        jnp.dot(w0t_ref[...], ft, preferred_element_type=jnp.float32)
        + b0_ref[...], 0.0)
    h = jnp.maximum(
        jnp.dot(w1t_ref[...], h, preferred_element_type=jnp.float32)
        + b1_ref[...], 0.0)
    mlp = jnp.dot(w2t_ref[...], h, preferred_element_type=jnp.float32)

    out_ref[...] = mlp + inter + lin + b2b_ref[...]


@jax.jit
def _tc_mlp(ft, wv, w0t, b0c, w1t, b1c, w2t, b2b):
    return pl.pallas_call(
        _tc_body,
        grid=(B // BBT,),
        in_specs=[
            pl.BlockSpec((FD, BBT), lambda i: (0, i)),
            pl.BlockSpec((F, BBT), lambda i: (0, i)),
            pl.BlockSpec((H0, FD), lambda i: (0, 0)),
            pl.BlockSpec((H0, 1), lambda i: (0, 0)),
            pl.BlockSpec((H1, H0), lambda i: (0, 0)),
            pl.BlockSpec((H1, 1), lambda i: (0, 0)),
            pl.BlockSpec((1, H1), lambda i: (0, 0)),
            pl.BlockSpec((1, 1), lambda i: (0, 0)),
        ],
        out_specs=pl.BlockSpec((1, BBT), lambda i: (0, i)),
        out_shape=jax.ShapeDtypeStruct((1, B), jnp.float32),
    )(ft, wv, w0t, b0c, w1t, b1c, w2t, b2b)


def kernel(x, emb, w_lin, b_lin, W0, b0, W1, b1, W2, b2):
    embt = emb.transpose(0, 2, 1).reshape(FD, V)
    xt = x.astype(jnp.int32).T.reshape(F * B)

    ft, wv = _sc_planes(embt, w_lin, xt)

    w0t = W0.reshape(D, F, H0).transpose(1, 0, 2).reshape(FD, H0).T
    b0c = b0.reshape(H0, 1)
    w1t = W1.T
    b1c = b1.reshape(H1, 1)
    w2t = W2.T
    b2b = (b2 + b_lin).reshape(1, 1)

    out = _tc_mlp(ft, wv, w0t, b0c, w1t, b1c, w2t, b2b)
    return out.reshape(B, 1)


# speedup vs baseline: 1.0036x; 1.0036x over previous
import jax
import jax.numpy as jnp
from jax import lax
from jax.experimental import pallas as pl
from jax.experimental.pallas import tpu as pltpu
from jax.experimental.pallas import tpu_sc as plsc

B = 16384
F = 26
V = 100000
D = 16
FD = F * D
H0 = 1024
H1 = 512

NC, NS, L = 2, 16, 16
NW = NC * NS
PPW = FD // NW
GCH = 4096


def _plane_body(embt, wlin, xt, featt, wvt, plane_v, xv, ob0, ob1,
                sem0, sem1):
    wid = lax.axis_index("s") * NC + lax.axis_index("c")
    obs, sems = (ob0, ob1), (sem0, sem1)
    pending = [None, None]

    def extract(fd, f, load_x, out_hbm, table_hbm):
        @pl.when(load_x)
        def _():
            pltpu.sync_copy(xt.at[pl.ds(f * B, B)], xv)

        pltpu.sync_copy(table_hbm.at[fd], plane_v)

        for c in range(B // GCH):
            slot = c % 2
            ob = obs[slot]
            if pending[slot] is not None:
                pending[slot].wait()

            @plsc.parallel_loop(0, GCH // L, unroll=8)
            def grp(g):
                idx = xv[pl.ds(c * GCH + g * L, L)]
                ob[pl.ds(g * L, L)] = plsc.load_gather(plane_v, [idx])
            cp = pltpu.make_async_copy(
                ob, out_hbm.at[fd, pl.ds(c * GCH, GCH)], sems[slot])
            cp.start()
            pending[slot] = cp

    def drain():
        for slot in range(2):
            if pending[slot] is not None:
                pending[slot].wait()
                pending[slot] = None

    for k in range(PPW):
        fd = wid * PPW + k
        load_x = (fd % D == 0) if k else (fd == fd)
        extract(fd, fd // D, load_x, featt, embt)
    drain()

    @pl.when(wid < F)
    def _():
        extract(wid, wid, wid == wid, wvt, wlin)
        drain()


@jax.jit
def _sc_planes(embt, wlin, xt):
    mesh = plsc.VectorSubcoreMesh(core_axis_name="c", subcore_axis_name="s")
    return pl.kernel(
        _plane_body,
        out_type=(
            jax.ShapeDtypeStruct((FD, B), jnp.float32),
            jax.ShapeDtypeStruct((F, B), jnp.float32),
        ),
        mesh=mesh,
        compiler_params=pltpu.CompilerParams(needs_layout_passes=False),
        scratch_types=[
            pltpu.VMEM((V,), jnp.float32),
            pltpu.VMEM((B,), jnp.int32),
            pltpu.VMEM((GCH,), jnp.float32),
            pltpu.VMEM((GCH,), jnp.float32),
            pltpu.SemaphoreType.DMA,
            pltpu.SemaphoreType.DMA,
        ],
        name="deepfm_plane_gather",
    )(embt, wlin, xt)


BBT = 4096


def _tc_body(ft_ref, wv_ref, w0t_ref, b0_ref, w1t_ref, b1_ref, w2t_ref,
             b2b_ref, out_ref):
    ft = ft_ref[...]
    sel = (lax.broadcasted_iota(jnp.int32, (D, FD), 1) % D
           == lax.broadcasted_iota(jnp.int32, (D, FD), 0)
           ).astype(jnp.float32)
    s = jnp.dot(sel, ft, preferred_element_type=jnp.float32)
    sq = jnp.dot(sel, ft * ft, preferred_element_type=jnp.float32)
    inter = 0.5 * jnp.sum(s * s - sq, axis=0, keepdims=True)

    lin = jnp.sum(wv_ref[...], axis=0, keepdims=True)

    h = jnp.maximum(
        jnp.dot(w0t_ref[...], ft.astype(jnp.bfloat16),
                preferred_element_type=jnp.float32)
        + b0_ref[...], 0.0)
    h = jnp.maximum(
        jnp.dot(w1t_ref[...], h.astype(jnp.bfloat16),
                preferred_element_type=jnp.float32)
        + b1_ref[...], 0.0)
    mlp = jnp.dot(w2t_ref[...], h, preferred_element_type=jnp.float32)

    out_ref[...] = mlp + inter + lin + b2b_ref[...]


@jax.jit
def _tc_mlp(ft, wv, w0t, b0c, w1t, b1c, w2t, b2b):
    return pl.pallas_call(
        _tc_body,
        grid=(B // BBT,),
        in_specs=[
            pl.BlockSpec((FD, BBT), lambda i: (0, i)),
            pl.BlockSpec((F, BBT), lambda i: (0, i)),
            pl.BlockSpec((H0, FD), lambda i: (0, 0)),
            pl.BlockSpec((H0, 1), lambda i: (0, 0)),
            pl.BlockSpec((H1, H0), lambda i: (0, 0)),
            pl.BlockSpec((H1, 1), lambda i: (0, 0)),
            pl.BlockSpec((1, H1), lambda i: (0, 0)),
            pl.BlockSpec((1, 1), lambda i: (0, 0)),
        ],
        out_specs=pl.BlockSpec((1, BBT), lambda i: (0, i)),
        out_shape=jax.ShapeDtypeStruct((1, B), jnp.float32),
    )(ft, wv, w0t, b0c, w1t, b1c, w2t, b2b)


def kernel(x, emb, w_lin, b_lin, W0, b0, W1, b1, W2, b2):
    embt = emb.transpose(0, 2, 1).reshape(FD, V)
    xt = x.astype(jnp.int32).T.reshape(F * B)

    ft, wv = _sc_planes(embt, w_lin, xt)

    w0t = W0.reshape(D, F, H0).transpose(1, 0, 2).reshape(FD, H0) \
        .T.astype(jnp.bfloat16)
    b0c = b0.reshape(H0, 1)
    w1t = W1.T.astype(jnp.bfloat16)
    b1c = b1.reshape(H1, 1)
    w2t = W2.T
    b2b = (b2 + b_lin).reshape(1, 1)

    out = _tc_mlp(ft, wv, w0t, b0c, w1t, b1c, w2t, b2b)
    return out.reshape(B, 1)
